# baseline (speedup 1.0000x reference)
import jax
import jax.numpy as jnp
from jax.experimental import pallas as pl

K = 1024
D = 64
B = 16
T = 576
CHUNK = 128
BATCH_BLK = 16
R = BATCH_BLK * T


def _vq_kernel(x_ref, w_ref, out_ref):
    x = x_ref[...].reshape(R, D)
    zz = jnp.sum(x * x, axis=1, keepdims=True)
    w = w_ref[...]
    w2 = 2.0 * w
    ww = jax.lax.dot_general(
        jnp.ones((1, D), jnp.float32), w * w,
        dimension_numbers=(((1,), (1,)), ((), ())),
        preferred_element_type=jnp.float32,
    )
    vks = []
    for k in range(K // CHUNK):
        w2k = w2[k * CHUNK:(k + 1) * CHUNK, :]
        wwk = ww[:, k * CHUNK:(k + 1) * CHUNK]
        dot2k = jax.lax.dot_general(
            x, w2k,
            dimension_numbers=(((1,), (1,)), ((), ())),
            preferred_element_type=jnp.float32,
        )
        vks.append(zz - dot2k + wwk)
    level = vks
    while len(level) > 1:
        level = [jnp.minimum(level[i], level[i + 1]) for i in range(0, len(level), 2)]
    m = jnp.min(level[0], axis=1, keepdims=True)
    cur = jnp.full((R, CHUNK), float(K // CHUNK), jnp.float32)
    for k in reversed(range(K // CHUNK)):
        cur = jnp.where(vks[k] == m, float(k), cur)
    lanef = jax.lax.broadcasted_iota(jnp.int32, (R, CHUNK), 1).astype(jnp.float32)
    cand = jnp.where(cur < float(K // CHUNK),
                     cur * float(CHUNK) + lanef, float(K))
    idxf = jnp.min(cand, axis=1, keepdims=True)
    idx = idxf.astype(jnp.int32)
    out_ref[...] = jax.lax.reshape(idx, (BATCH_BLK, T))


def kernel(z_e_x, weight):
    return pl.pallas_call(
        _vq_kernel,
        grid=(B // BATCH_BLK,),
        in_specs=[
            pl.BlockSpec((BATCH_BLK, T, D), lambda i: (i, 0, 0)),
            pl.BlockSpec((K, D), lambda i: (0, 0)),
        ],
        out_specs=pl.BlockSpec((BATCH_BLK, T), lambda i: (i, 0)),
        out_shape=jax.ShapeDtypeStruct((B, T), jnp.int32),
    )(z_e_x, weight)


# speedup vs baseline: 1.0372x; 1.0372x over previous
import jax
import jax.numpy as jnp
from jax.experimental import pallas as pl

K = 1024
D = 64
B = 16
T = 576
CHUNK = 256
BATCH_BLK = 8
R = BATCH_BLK * T


def _vq_kernel(x_ref, w_ref, out_ref):
    x = x_ref[...].reshape(R, D)
    zz = jnp.sum(x * x, axis=1, keepdims=True)
    w = w_ref[...]
    w2 = 2.0 * w
    ww = jax.lax.dot_general(
        jnp.ones((1, D), jnp.float32), w * w,
        dimension_numbers=(((1,), (1,)), ((), ())),
        preferred_element_type=jnp.float32,
    )
    vks = []
    for k in range(K // CHUNK):
        w2k = w2[k * CHUNK:(k + 1) * CHUNK, :]
        wwk = ww[:, k * CHUNK:(k + 1) * CHUNK]
        dot2k = jax.lax.dot_general(
            x, w2k,
            dimension_numbers=(((1,), (1,)), ((), ())),
            preferred_element_type=jnp.float32,
        )
        vks.append(zz - dot2k + wwk)
    level = vks
    while len(level) > 1:
        level = [jnp.minimum(level[i], level[i + 1]) for i in range(0, len(level), 2)]
    m = jnp.min(level[0], axis=1, keepdims=True)
    cur = jnp.full((R, CHUNK), float(K // CHUNK), jnp.float32)
    for k in reversed(range(K // CHUNK)):
        cur = jnp.where(vks[k] == m, float(k), cur)
    lanef = jax.lax.broadcasted_iota(jnp.int32, (R, CHUNK), 1).astype(jnp.float32)
    cand = jnp.where(cur < float(K // CHUNK),
                     cur * float(CHUNK) + lanef, float(K))
    idxf = jnp.min(cand, axis=1, keepdims=True)
    idx = idxf.astype(jnp.int32)
    out_ref[...] = jax.lax.reshape(idx, (BATCH_BLK, T))


def kernel(z_e_x, weight):
    return pl.pallas_call(
        _vq_kernel,
        grid=(B // BATCH_BLK,),
        in_specs=[
            pl.BlockSpec((BATCH_BLK, T, D), lambda i: (i, 0, 0)),
            pl.BlockSpec((K, D), lambda i: (0, 0)),
        ],
        out_specs=pl.BlockSpec((BATCH_BLK, T), lambda i: (i, 0)),
        out_shape=jax.ShapeDtypeStruct((B, T), jnp.int32),
    )(z_e_x, weight)


# speedup vs baseline: 1.0581x; 1.0201x over previous
import jax
import jax.numpy as jnp
from jax.experimental import pallas as pl

K = 1024
D = 64
B = 16
T = 576
CHUNK = 256
BATCH_BLK = 8
R = BATCH_BLK * T


def _vq_kernel(x_ref, w_ref, out_ref):
    x = x_ref[...].reshape(R, D)
    zz = jnp.sum(x * x, axis=1, keepdims=True)
    w = w_ref[...]
    w2 = 2.0 * w
    ww = jax.lax.dot_general(
        jnp.ones((1, D), jnp.float32), w * w,
        dimension_numbers=(((1,), (1,)), ((), ())),
        preferred_element_type=jnp.float32,
    )
    vks = []
    for k in range(K // CHUNK):
        w2k = w2[k * CHUNK:(k + 1) * CHUNK, :]
        wwk = ww[:, k * CHUNK:(k + 1) * CHUNK]
        dot2k = jax.lax.dot_general(
            x, w2k,
            dimension_numbers=(((1,), (1,)), ((), ())),
            preferred_element_type=jnp.float32,
        )
        vks.append(zz - dot2k + wwk)
    level = vks
    while len(level) > 1:
        level = [jnp.minimum(level[i], level[i + 1]) for i in range(0, len(level), 2)]
    m = jnp.min(level[0], axis=1, keepdims=True)
    cur = jnp.full((R, CHUNK), float(K // CHUNK), jnp.float32)
    for k in reversed(range(K // CHUNK)):
        cur = jnp.where(vks[k] == m, float(k), cur)
    lanef = jax.lax.broadcasted_iota(jnp.int32, (R, CHUNK), 1).astype(jnp.float32)
    cand = cur * float(CHUNK) + lanef
    idxf = jnp.min(cand, axis=1, keepdims=True)
    idx = idxf.astype(jnp.int32)
    out_ref[...] = jax.lax.reshape(idx, (BATCH_BLK, T))


def kernel(z_e_x, weight):
    return pl.pallas_call(
        _vq_kernel,
        grid=(B // BATCH_BLK,),
        in_specs=[
            pl.BlockSpec((BATCH_BLK, T, D), lambda i: (i, 0, 0)),
            pl.BlockSpec((K, D), lambda i: (0, 0)),
        ],
        out_specs=pl.BlockSpec((BATCH_BLK, T), lambda i: (i, 0)),
        out_shape=jax.ShapeDtypeStruct((B, T), jnp.int32),
    )(z_e_x, weight)
